# Initial kernel scaffold; baseline (speedup 1.0000x reference)
import jax
import jax.numpy as jnp
from jax.experimental import pallas as pl


def kernel(edge_feats, segment_ids, W, b):
    raise NotImplementedError("write your pallas kernel here")


# speedup vs baseline: 4.2161x; 4.2161x over previous
import functools

import jax
import jax.numpy as jnp
from jax.experimental import pallas as pl
from jax.experimental.pallas import tpu as pltpu

E = 320000
D = 128
G = 256
B = 2560
NB = E // B

NEG_INF = float("-inf")


def _body(lo_ref, hi_ref, ids_ref, x_ref, w_ref, b_ref, out_ref,
          sum_acc, max_acc):
    i = pl.program_id(0)

    @pl.when(i == 0)
    def _init():
        sum_acc[...] = jnp.zeros_like(sum_acc)
        max_acc[...] = jnp.full_like(max_acc, NEG_INF)

    x = x_ref[...]
    wv = jnp.tanh(
        jax.lax.dot_general(x, w_ref[...], (((1,), (0,)), ((), ())),
                            preferred_element_type=jnp.float32)
        + b_ref[0, 0])
    xw = x * wv
    ids = ids_ref[0]

    lo = lo_ref[i]
    hi = hi_ref[i]

    def seg_body(g, carry):
        m = ids == g
        s = jnp.sum(jnp.where(m, xw, 0.0), axis=0, keepdims=True)
        mx = jnp.max(jnp.where(m, x, NEG_INF), axis=0, keepdims=True)
        sum_acc[pl.ds(g, 1), :] += s
        max_acc[pl.ds(g, 1), :] = jnp.maximum(max_acc[pl.ds(g, 1), :], mx)
        return carry

    jax.lax.fori_loop(lo, hi + 1, seg_body, 0)

    @pl.when(i == NB - 1)
    def _emit():
        out_ref[:, :D] = sum_acc[...]
        out_ref[:, D:] = max_acc[...]


@jax.jit
def kernel(edge_feats, segment_ids, W, b):
    ids32 = segment_ids.astype(jnp.int32)
    ids3 = ids32.reshape(NB, B, 1)
    blk = ids32.reshape(NB, B)
    lo = blk[:, 0]
    hi = blk[:, -1]
    b2 = b.reshape(1, 1).astype(jnp.float32)

    grid_spec = pltpu.PrefetchScalarGridSpec(
        num_scalar_prefetch=2,
        grid=(NB,),
        in_specs=[
            pl.BlockSpec((1, B, 1), lambda i, lo_r, hi_r: (i, 0, 0)),
            pl.BlockSpec((B, D), lambda i, lo_r, hi_r: (i, 0)),
            pl.BlockSpec((D, 1), lambda i, lo_r, hi_r: (0, 0)),
            pl.BlockSpec((1, 1), lambda i, lo_r, hi_r: (0, 0)),
        ],
        out_specs=pl.BlockSpec((G, 2 * D), lambda i, lo_r, hi_r: (0, 0)),
        scratch_shapes=[
            pltpu.VMEM((G, D), jnp.float32),
            pltpu.VMEM((G, D), jnp.float32),
        ],
    )
    return pl.pallas_call(
        _body,
        grid_spec=grid_spec,
        out_shape=jax.ShapeDtypeStruct((G, 2 * D), jnp.float32),
    )(lo, hi, ids3, edge_feats, W.astype(jnp.float32), b2)


# speedup vs baseline: 4.3334x; 1.0278x over previous
import functools

import jax
import jax.numpy as jnp
from jax import lax
from jax.experimental import pallas as pl
from jax.experimental.pallas import tpu as pltpu
from jax.experimental.pallas import tpu_sc as plsc

E = 320000
D = 128
G = 256
L = 16
NC = 2
NS = 16
NW = NC * NS
SPW = G // NW
C = 256
NV = D // L
OFF_PAD = G + L

NEG_INF = float("-inf")


def _sc_body(x_hbm, off_hbm, w_hbm, b_hbm, out_hbm, buf, wv_ref, bv_ref,
             off_ref, stage):
    cid = lax.axis_index("c")
    sid = lax.axis_index("s")
    wid = sid * NC + cid
    g0 = wid * SPW

    pltpu.sync_copy(w_hbm, wv_ref)
    pltpu.sync_copy(b_hbm, bv_ref)
    pltpu.sync_copy(off_hbm, off_ref)

    wvec = [wv_ref[pl.ds(L * k, L)] for k in range(NV)]
    bv = bv_ref[...]
    iota = lax.iota(jnp.int32, L)

    def lane_shuffle(v, idx):
        return lax.gather(
            v, idx[:, None],
            lax.GatherDimensionNumbers(
                offset_dims=(), collapsed_slice_dims=(0,),
                start_index_map=(0,)),
            slice_sizes=(1,),
            mode=lax.GatherScatterMode.PROMISE_IN_BOUNDS)

    def off_at(idx):
        return off_ref[pl.ds(idx, L)][0]

    for j in range(SPW):
        e0 = off_at(g0 + j)
        e1 = off_at(g0 + j + 1)
        acc0 = (tuple(jnp.zeros((L,), jnp.float32) for _ in range(NV))
                + tuple(jnp.full((L,), NEG_INF, jnp.float32)
                        for _ in range(NV)))
        nch = (e1 - e0 + (C - 1)) // C

        def chunk_body(ci, acc, e0=e0, e1=e1):
            start = e0 + ci * C
            m = jnp.minimum(C, e1 - start)
            s_dma = jnp.minimum(start, E - C)
            j0 = start - s_dma
            pltpu.sync_copy(x_hbm.at[pl.ds(s_dma * D, C * D)], buf)

            def row_body(r, a):
                base = (j0 + r) * D
                xs = [buf[pl.ds(base + L * k, L)] for k in range(NV)]
                p = xs[0] * wvec[0]
                for k in range(1, NV):
                    p = p + xs[k] * wvec[k]
                for sh in (8, 4, 2, 1):
                    p = p + lane_shuffle(p, iota ^ sh)
                z = p + bv
                ex = jnp.exp(-2.0 * jnp.abs(z))
                wgt = jnp.sign(z) * (1.0 - ex) / (1.0 + ex)
                news = tuple(a[k] + xs[k] * wgt for k in range(NV))
                newm = tuple(jnp.maximum(a[NV + k], xs[k]) for k in range(NV))
                return news + newm

            return plsc.parallel_loop(0, m, unroll=1, carry=acc)(row_body)

        acc = lax.fori_loop(0, nch, chunk_body, acc0)
        for k in range(NV):
            stage[j, pl.ds(L * k, L)] = acc[k]
            stage[j, pl.ds(D + L * k, L)] = acc[NV + k]

    pltpu.sync_copy(stage, out_hbm.at[pl.ds(g0, SPW), :])


@jax.jit
def kernel(edge_feats, segment_ids, W, b):
    ids32 = segment_ids.astype(jnp.int32)
    offsets = jnp.searchsorted(
        ids32, jnp.arange(G + 1, dtype=jnp.int32), side="left"
    ).astype(jnp.int32)
    off_pad = jnp.concatenate(
        [offsets, jnp.zeros((OFF_PAD - (G + 1),), jnp.int32)])
    x_flat = edge_feats.reshape(E * D)
    w_flat = W.reshape(D).astype(jnp.float32)
    b16 = jnp.broadcast_to(b.astype(jnp.float32), (L,))

    mesh = plsc.VectorSubcoreMesh(
        core_axis_name="c", subcore_axis_name="s",
        num_cores=NC, num_subcores=NS)
    f = pl.kernel(
        _sc_body,
        out_type=jax.ShapeDtypeStruct((G, 2 * D), jnp.float32),
        mesh=mesh,
        scratch_types=[
            pltpu.VMEM((C * D,), jnp.float32),
            pltpu.VMEM((D,), jnp.float32),
            pltpu.VMEM((L,), jnp.float32),
            pltpu.VMEM((OFF_PAD,), jnp.int32),
            pltpu.VMEM((SPW, 2 * D), jnp.float32),
        ],
    )
    return f(x_flat, off_pad, w_flat, b16)


# speedup vs baseline: 4.8670x; 1.1231x over previous
import functools

import jax
import jax.numpy as jnp
from jax import lax
from jax.experimental import pallas as pl
from jax.experimental.pallas import tpu as pltpu
from jax.experimental.pallas import tpu_sc as plsc

E = 320000
D = 128
G = 256
L = 16
NC = 2
NS = 16
NW = NC * NS
SPW = G // NW
C = 256
NV = D // L
OFF_PAD = G + L

NEG_INF = float("-inf")


def _sc_body(x_hbm, off_hbm, w_hbm, b_hbm, out_hbm, buf, wv_ref, bv_ref,
             off_ref, stage, sem0, sem1):
    cid = lax.axis_index("c")
    sid = lax.axis_index("s")
    wid = sid * NC + cid
    g0 = wid * SPW

    pltpu.sync_copy(w_hbm, wv_ref)
    pltpu.sync_copy(b_hbm, bv_ref)
    pltpu.sync_copy(off_hbm, off_ref)

    wvec = [wv_ref[pl.ds(L * k, L)] for k in range(NV)]
    bv = bv_ref[...]
    iota = lax.iota(jnp.int32, L)

    def lane_shuffle(v, idx):
        return lax.gather(
            v, idx[:, None],
            lax.GatherDimensionNumbers(
                offset_dims=(), collapsed_slice_dims=(0,),
                start_index_map=(0,)),
            slice_sizes=(1,),
            mode=lax.GatherScatterMode.PROMISE_IN_BOUNDS)

    def off_at(idx):
        return off_ref[pl.ds(idx, L)][0]

    CD = C * D
    sems = (sem0, sem1)

    for j in range(SPW):
        e0 = off_at(g0 + j)
        e1 = off_at(g0 + j + 1)
        acc0 = (tuple(jnp.zeros((L,), jnp.float32) for _ in range(NV))
                + tuple(jnp.full((L,), NEG_INF, jnp.float32)
                        for _ in range(NV)))
        nch = (e1 - e0 + (C - 1)) // C
        npair = (nch + 1) // 2

        def issue(ci, par, e0=e0, nch=nch):
            @pl.when(ci < nch)
            def _():
                start = e0 + ci * C
                s_dma = jnp.minimum(start, E - C)
                pltpu.async_copy(
                    x_hbm.at[pl.ds(s_dma * D, CD)],
                    buf.at[pl.ds(par * CD, CD)], sems[par])

        def compute(ci, par, acc, e0=e0, e1=e1, nch=nch):
            start = e0 + ci * C
            m = jnp.maximum(0, jnp.minimum(C, e1 - start))
            s_dma = jnp.minimum(start, E - C)
            j0 = start - s_dma

            @pl.when(ci < nch)
            def _():
                pltpu.make_async_copy(
                    x_hbm.at[pl.ds(0, CD)],
                    buf.at[pl.ds(par * CD, CD)], sems[par]).wait()

            def row_body(r, a):
                base = par * CD + (j0 + r) * D
                xs = [buf[pl.ds(base + L * k, L)] for k in range(NV)]
                p = xs[0] * wvec[0]
                for k in range(1, NV):
                    p = p + xs[k] * wvec[k]
                for sh in (8, 4, 2, 1):
                    p = p + lane_shuffle(p, iota ^ sh)
                z = p + bv
                ex = jnp.exp(-2.0 * jnp.abs(z))
                wgt = jnp.sign(z) * (1.0 - ex) / (1.0 + ex)
                news = tuple(a[k] + xs[k] * wgt for k in range(NV))
                newm = tuple(jnp.maximum(a[NV + k], xs[k]) for k in range(NV))
                return news + newm

            return plsc.parallel_loop(0, m, unroll=4, carry=acc)(row_body)

        issue(0, 0)

        def pair_body(t, acc):
            ci = 2 * t
            issue(ci + 1, 1)
            acc = compute(ci, 0, acc)
            issue(ci + 2, 0)
            acc = compute(ci + 1, 1, acc)
            return acc

        acc = lax.fori_loop(0, npair, pair_body, acc0)
        for k in range(NV):
            stage[j, pl.ds(L * k, L)] = acc[k]
            stage[j, pl.ds(D + L * k, L)] = acc[NV + k]

    pltpu.sync_copy(stage, out_hbm.at[pl.ds(g0, SPW), :])


@jax.jit
def kernel(edge_feats, segment_ids, W, b):
    ids32 = segment_ids.astype(jnp.int32)
    offsets = jnp.searchsorted(
        ids32, jnp.arange(G + 1, dtype=jnp.int32), side="left"
    ).astype(jnp.int32)
    off_pad = jnp.concatenate(
        [offsets, jnp.zeros((OFF_PAD - (G + 1),), jnp.int32)])
    x_flat = edge_feats.reshape(E * D)
    w_flat = W.reshape(D).astype(jnp.float32)
    b16 = jnp.broadcast_to(b.astype(jnp.float32), (L,))

    mesh = plsc.VectorSubcoreMesh(
        core_axis_name="c", subcore_axis_name="s",
        num_cores=NC, num_subcores=NS)
    f = pl.kernel(
        _sc_body,
        out_type=jax.ShapeDtypeStruct((G, 2 * D), jnp.float32),
        mesh=mesh,
        scratch_types=[
            pltpu.VMEM((2 * C * D,), jnp.float32),
            pltpu.VMEM((D,), jnp.float32),
            pltpu.VMEM((L,), jnp.float32),
            pltpu.VMEM((OFF_PAD,), jnp.int32),
            pltpu.VMEM((SPW, 2 * D), jnp.float32),
            pltpu.SemaphoreType.DMA,
            pltpu.SemaphoreType.DMA,
        ],
    )
    return f(x_flat, off_pad, w_flat, b16)
